# baseline (speedup 1.0000x reference)
import jax
import jax.numpy as jnp
from jax.experimental import pallas as pl
from jax.experimental.pallas import tpu as pltpu

B, N, D, E = 8, 512, 32, 8
NN = N * N
MB = 8
L = MB * N
NBLK = N // MB
BF = jnp.bfloat16


def _layer_pass(X_ref, m_ref, Wm_ref, h2T, k_rows, write_edges, residual_edges):
    def dot_blk(bi):
        Xb = X_ref[0:k_rows, pl.ds(bi * L, L)]
        return jnp.dot(Wm_ref[...], Xb, preferred_element_type=jnp.float32)

    def consume(bi, G):
        cols = [jnp.sum(jnp.maximum(G[0:D, i * N:(i + 1) * N], 0.0) * h2T,
                        axis=1, keepdims=True)
                for i in range(MB)]
        m_ref[pl.ds(bi * MB, MB), :] = jnp.concatenate(cols, axis=1).T
        if write_edges:
            eg = jnp.maximum(G[D:D + E], 0.0).astype(BF)
            if residual_edges:
                eg = X_ref[E:2 * E, pl.ds(bi * L, L)] + eg
            X_ref[E:2 * E, pl.ds(bi * L, L)] = eg

    def body(bi, Gprev):
        Gnew = dot_blk(bi)
        consume(bi - 1, Gprev)
        return Gnew

    Glast = jax.lax.fori_loop(1, NBLK, body, dot_blk(0))
    consume(NBLK - 1, Glast)


def _body(scaff_ref, z_ref,
          W1_0r, W2_0r, Wm0r,
          W1_1r, W2_1r, Wm1r,
          W1_2r, W2_2r, Wm2r,
          x_out_ref, X_ref, m_ref):
    X_ref[0:E, :] = scaff_ref[0]
    x = z_ref[0]
    h2T = jnp.dot(x, W2_0r[...], preferred_element_type=jnp.float32).T
    _layer_pass(X_ref, m_ref, Wm0r, h2T, E, True, False)
    x = jnp.maximum(jnp.dot(x, W1_0r[...], preferred_element_type=jnp.float32)
                    + m_ref[...], 0.0)
    h2T = jnp.dot(x, W2_1r[...], preferred_element_type=jnp.float32).T
    _layer_pass(X_ref, m_ref, Wm1r, h2T, 2 * E, True, True)
    x = x + jnp.maximum(jnp.dot(x, W1_1r[...], preferred_element_type=jnp.float32)
                        + m_ref[...], 0.0)
    h2T = jnp.dot(x, W2_2r[...], preferred_element_type=jnp.float32).T
    _layer_pass(X_ref, m_ref, Wm2r, h2T, 2 * E, False, False)
    x_out_ref[0] = jnp.maximum(
        jnp.dot(x, W1_2r[...], preferred_element_type=jnp.float32) + m_ref[...], 0.0)


def kernel(z, scaffold, W1_0, W2_0, Wg_0, We_0,
           W1_1, W2_1, Wge_1, Wgs_1, We1_1, We2_1,
           W1_2, W2_2, Wge_2, Wgs_2, We1_2, We2_2):
    scaff = scaffold.reshape(B, E, NN).astype(BF)
    Wm0 = jnp.concatenate([Wg_0.T, We_0.T], axis=0).astype(BF)
    Wm1 = jnp.concatenate(
        [jnp.concatenate([Wgs_1.T, Wge_1.T], axis=1),
         jnp.concatenate([We2_1.T, We1_1.T], axis=1)], axis=0).astype(BF)
    Wm2 = jnp.concatenate([Wgs_2.T, Wge_2.T], axis=1).astype(BF)

    full = lambda s: pl.BlockSpec(s, lambda b: tuple(0 for _ in s))
    in_specs = [
        pl.BlockSpec((1, E, NN), lambda b: (b, 0, 0)),
        pl.BlockSpec((1, N, D), lambda b: (b, 0, 0)),
        full((D, D)), full((D, D)), full((D + E, E)),
        full((D, D)), full((D, D)), full((D + E, 2 * E)),
        full((D, D)), full((D, D)), full((D, 2 * E)),
    ]
    out = pl.pallas_call(
        _body,
        grid=(B,),
        in_specs=in_specs,
        out_specs=pl.BlockSpec((1, N, D), lambda b: (b, 0, 0)),
        out_shape=jax.ShapeDtypeStruct((B, N, D), jnp.float32),
        scratch_shapes=[
            pltpu.VMEM((2 * E, NN), BF),
            pltpu.VMEM((N, D), jnp.float32),
        ],
        compiler_params=pltpu.CompilerParams(
            dimension_semantics=("parallel",),
        ),
    )(scaff, z, W1_0, W2_0, Wm0,
      W1_1, W2_1, Wm1,
      W1_2, W2_2, Wm2)
    return out


# speedup vs baseline: 1.2060x; 1.2060x over previous
import jax
import jax.numpy as jnp
from jax.experimental import pallas as pl
from jax.experimental.pallas import tpu as pltpu

B, N, D, E = 8, 512, 32, 8
NN = N * N
MB = 64
L = MB * N
NBLK = N // MB
BF = jnp.bfloat16


def _layer_pass(X_ref, m_ref, Wm_ref, h2T, k_rows, write_edges, residual_edges):
    def dot_blk(bi):
        Xb = X_ref[0:k_rows, pl.ds(bi * L, L)]
        return jnp.dot(Wm_ref[...], Xb, preferred_element_type=jnp.float32)

    def consume(bi, G):
        cols = [jnp.sum(jnp.maximum(G[0:D, i * N:(i + 1) * N], 0.0) * h2T,
                        axis=1, keepdims=True)
                for i in range(MB)]
        m_ref[pl.ds(bi * MB, MB), :] = jnp.concatenate(cols, axis=1).T
        if write_edges:
            eg = jnp.maximum(G[D:D + E], 0.0).astype(BF)
            if residual_edges:
                eg = X_ref[E:2 * E, pl.ds(bi * L, L)] + eg
            X_ref[E:2 * E, pl.ds(bi * L, L)] = eg

    def body(bi, Gprev):
        Gnew = dot_blk(bi)
        consume(bi - 1, Gprev)
        return Gnew

    Glast = jax.lax.fori_loop(1, NBLK, body, dot_blk(0))
    consume(NBLK - 1, Glast)


def _body(scaff_ref, z_ref,
          W1_0r, W2_0r, Wm0r,
          W1_1r, W2_1r, Wm1r,
          W1_2r, W2_2r, Wm2r,
          x_out_ref, X_ref, m_ref):
    X_ref[0:E, :] = scaff_ref[0]
    x = z_ref[0]
    h2T = jnp.dot(x, W2_0r[...], preferred_element_type=jnp.float32).T
    _layer_pass(X_ref, m_ref, Wm0r, h2T, E, True, False)
    x = jnp.maximum(jnp.dot(x, W1_0r[...], preferred_element_type=jnp.float32)
                    + m_ref[...], 0.0)
    h2T = jnp.dot(x, W2_1r[...], preferred_element_type=jnp.float32).T
    _layer_pass(X_ref, m_ref, Wm1r, h2T, 2 * E, True, True)
    x = x + jnp.maximum(jnp.dot(x, W1_1r[...], preferred_element_type=jnp.float32)
                        + m_ref[...], 0.0)
    h2T = jnp.dot(x, W2_2r[...], preferred_element_type=jnp.float32).T
    _layer_pass(X_ref, m_ref, Wm2r, h2T, 2 * E, False, False)
    x_out_ref[0] = jnp.maximum(
        jnp.dot(x, W1_2r[...], preferred_element_type=jnp.float32) + m_ref[...], 0.0)


def kernel(z, scaffold, W1_0, W2_0, Wg_0, We_0,
           W1_1, W2_1, Wge_1, Wgs_1, We1_1, We2_1,
           W1_2, W2_2, Wge_2, Wgs_2, We1_2, We2_2):
    scaff = scaffold.reshape(B, E, NN).astype(BF)
    Wm0 = jnp.concatenate([Wg_0.T, We_0.T], axis=0).astype(BF)
    Wm1 = jnp.concatenate(
        [jnp.concatenate([Wgs_1.T, Wge_1.T], axis=1),
         jnp.concatenate([We2_1.T, We1_1.T], axis=1)], axis=0).astype(BF)
    Wm2 = jnp.concatenate([Wgs_2.T, Wge_2.T], axis=1).astype(BF)

    full = lambda s: pl.BlockSpec(s, lambda b: tuple(0 for _ in s))
    in_specs = [
        pl.BlockSpec((1, E, NN), lambda b: (b, 0, 0)),
        pl.BlockSpec((1, N, D), lambda b: (b, 0, 0)),
        full((D, D)), full((D, D)), full((D + E, E)),
        full((D, D)), full((D, D)), full((D + E, 2 * E)),
        full((D, D)), full((D, D)), full((D, 2 * E)),
    ]
    out = pl.pallas_call(
        _body,
        grid=(B,),
        in_specs=in_specs,
        out_specs=pl.BlockSpec((1, N, D), lambda b: (b, 0, 0)),
        out_shape=jax.ShapeDtypeStruct((B, N, D), jnp.float32),
        scratch_shapes=[
            pltpu.VMEM((2 * E, NN), BF),
            pltpu.VMEM((N, D), jnp.float32),
        ],
        compiler_params=pltpu.CompilerParams(
            dimension_semantics=("parallel",),
        ),
    )(scaff, z, W1_0, W2_0, Wm0,
      W1_1, W2_1, Wm1,
      W1_2, W2_2, Wm2)
    return out


# speedup vs baseline: 1.2240x; 1.0150x over previous
import jax
import jax.numpy as jnp
from jax.experimental import pallas as pl
from jax.experimental.pallas import tpu as pltpu

B, N, D, E = 8, 512, 32, 8
NN = N * N
MB = 128
L = MB * N
NBLK = N // MB
BF = jnp.bfloat16


def _layer_pass(X_ref, m_ref, Wm_ref, h2T, k_rows, write_edges, residual_edges):
    def dot_blk(bi):
        Xb = X_ref[0:k_rows, pl.ds(bi * L, L)]
        return jnp.dot(Wm_ref[...], Xb, preferred_element_type=jnp.float32)

    def consume(bi, G):
        cols = [jnp.sum(jnp.maximum(G[0:D, i * N:(i + 1) * N], 0.0) * h2T,
                        axis=1, keepdims=True)
                for i in range(MB)]
        m_ref[pl.ds(bi * MB, MB), :] = jnp.concatenate(cols, axis=1).T
        if write_edges:
            eg = jnp.maximum(G[D:D + E], 0.0).astype(BF)
            if residual_edges:
                eg = X_ref[E:2 * E, pl.ds(bi * L, L)] + eg
            X_ref[E:2 * E, pl.ds(bi * L, L)] = eg

    def body(bi, Gprev):
        Gnew = dot_blk(bi)
        consume(bi - 1, Gprev)
        return Gnew

    Glast = jax.lax.fori_loop(1, NBLK, body, dot_blk(0))
    consume(NBLK - 1, Glast)


def _body(scaff_ref, z_ref,
          W1_0r, W2_0r, Wm0r,
          W1_1r, W2_1r, Wm1r,
          W1_2r, W2_2r, Wm2r,
          x_out_ref, X_ref, m_ref):
    X_ref[0:E, :] = scaff_ref[0]
    x = z_ref[0]
    h2T = jnp.dot(x, W2_0r[...], preferred_element_type=jnp.float32).T
    _layer_pass(X_ref, m_ref, Wm0r, h2T, E, True, False)
    x = jnp.maximum(jnp.dot(x, W1_0r[...], preferred_element_type=jnp.float32)
                    + m_ref[...], 0.0)
    h2T = jnp.dot(x, W2_1r[...], preferred_element_type=jnp.float32).T
    _layer_pass(X_ref, m_ref, Wm1r, h2T, 2 * E, True, True)
    x = x + jnp.maximum(jnp.dot(x, W1_1r[...], preferred_element_type=jnp.float32)
                        + m_ref[...], 0.0)
    h2T = jnp.dot(x, W2_2r[...], preferred_element_type=jnp.float32).T
    _layer_pass(X_ref, m_ref, Wm2r, h2T, 2 * E, False, False)
    x_out_ref[0] = jnp.maximum(
        jnp.dot(x, W1_2r[...], preferred_element_type=jnp.float32) + m_ref[...], 0.0)


def kernel(z, scaffold, W1_0, W2_0, Wg_0, We_0,
           W1_1, W2_1, Wge_1, Wgs_1, We1_1, We2_1,
           W1_2, W2_2, Wge_2, Wgs_2, We1_2, We2_2):
    scaff = scaffold.reshape(B, E, NN).astype(BF)
    Wm0 = jnp.concatenate([Wg_0.T, We_0.T], axis=0).astype(BF)
    Wm1 = jnp.concatenate(
        [jnp.concatenate([Wgs_1.T, Wge_1.T], axis=1),
         jnp.concatenate([We2_1.T, We1_1.T], axis=1)], axis=0).astype(BF)
    Wm2 = jnp.concatenate([Wgs_2.T, Wge_2.T], axis=1).astype(BF)

    full = lambda s: pl.BlockSpec(s, lambda b: tuple(0 for _ in s))
    in_specs = [
        pl.BlockSpec((1, E, NN), lambda b: (b, 0, 0)),
        pl.BlockSpec((1, N, D), lambda b: (b, 0, 0)),
        full((D, D)), full((D, D)), full((D + E, E)),
        full((D, D)), full((D, D)), full((D + E, 2 * E)),
        full((D, D)), full((D, D)), full((D, 2 * E)),
    ]
    out = pl.pallas_call(
        _body,
        grid=(B,),
        in_specs=in_specs,
        out_specs=pl.BlockSpec((1, N, D), lambda b: (b, 0, 0)),
        out_shape=jax.ShapeDtypeStruct((B, N, D), jnp.float32),
        scratch_shapes=[
            pltpu.VMEM((2 * E, NN), BF),
            pltpu.VMEM((N, D), jnp.float32),
        ],
        compiler_params=pltpu.CompilerParams(
            dimension_semantics=("parallel",),
        ),
    )(scaff, z, W1_0, W2_0, Wm0,
      W1_1, W2_1, Wm1,
      W1_2, W2_2, Wm2)
    return out


# speedup vs baseline: 2.0456x; 1.6712x over previous
import jax
import jax.numpy as jnp
from jax.experimental import pallas as pl
from jax.experimental.pallas import tpu as pltpu

B, N, D, E = 8, 512, 32, 8
NN = N * N
MB = 256
L = MB * N
NBLK = N // MB
BF = jnp.bfloat16


def _layer_pass(X_ref, m_ref, Wm_ref, h2T, k_rows, write_edges, residual_edges):
    def dot_blk(bi):
        Xb = X_ref[0:k_rows, pl.ds(bi * L, L)]
        return jnp.dot(Wm_ref[...], Xb, preferred_element_type=jnp.float32)

    def consume(bi, G):
        cols = [jnp.sum(jnp.maximum(G[0:D, i * N:(i + 1) * N], 0.0) * h2T,
                        axis=1, keepdims=True)
                for i in range(MB)]
        m_ref[pl.ds(bi * MB, MB), :] = jnp.concatenate(cols, axis=1).T
        if write_edges:
            eg = jnp.maximum(G[D:D + E], 0.0).astype(BF)
            if residual_edges:
                eg = X_ref[E:2 * E, pl.ds(bi * L, L)] + eg
            X_ref[E:2 * E, pl.ds(bi * L, L)] = eg

    def body(bi, Gprev):
        Gnew = dot_blk(bi)
        consume(bi - 1, Gprev)
        return Gnew

    Glast = jax.lax.fori_loop(1, NBLK, body, dot_blk(0))
    consume(NBLK - 1, Glast)


def _body(scaff_ref, z_ref,
          W1_0r, W2_0r, Wm0r,
          W1_1r, W2_1r, Wm1r,
          W1_2r, W2_2r, Wm2r,
          x_out_ref, X_ref, m_ref):
    X_ref[0:E, :] = scaff_ref[0]
    x = z_ref[0]
    h2T = jnp.dot(x, W2_0r[...], preferred_element_type=jnp.float32).T
    _layer_pass(X_ref, m_ref, Wm0r, h2T, E, True, False)
    x = jnp.maximum(jnp.dot(x, W1_0r[...], preferred_element_type=jnp.float32)
                    + m_ref[...], 0.0)
    h2T = jnp.dot(x, W2_1r[...], preferred_element_type=jnp.float32).T
    _layer_pass(X_ref, m_ref, Wm1r, h2T, 2 * E, True, True)
    x = x + jnp.maximum(jnp.dot(x, W1_1r[...], preferred_element_type=jnp.float32)
                        + m_ref[...], 0.0)
    h2T = jnp.dot(x, W2_2r[...], preferred_element_type=jnp.float32).T
    _layer_pass(X_ref, m_ref, Wm2r, h2T, 2 * E, False, False)
    x_out_ref[0] = jnp.maximum(
        jnp.dot(x, W1_2r[...], preferred_element_type=jnp.float32) + m_ref[...], 0.0)


def kernel(z, scaffold, W1_0, W2_0, Wg_0, We_0,
           W1_1, W2_1, Wge_1, Wgs_1, We1_1, We2_1,
           W1_2, W2_2, Wge_2, Wgs_2, We1_2, We2_2):
    scaff = scaffold.reshape(B, E, NN).astype(BF)
    Wm0 = jnp.concatenate([Wg_0.T, We_0.T], axis=0).astype(BF)
    Wm1 = jnp.concatenate(
        [jnp.concatenate([Wgs_1.T, Wge_1.T], axis=1),
         jnp.concatenate([We2_1.T, We1_1.T], axis=1)], axis=0).astype(BF)
    Wm2 = jnp.concatenate([Wgs_2.T, Wge_2.T], axis=1).astype(BF)

    full = lambda s: pl.BlockSpec(s, lambda b: tuple(0 for _ in s))
    in_specs = [
        pl.BlockSpec((1, E, NN), lambda b: (b, 0, 0)),
        pl.BlockSpec((1, N, D), lambda b: (b, 0, 0)),
        full((D, D)), full((D, D)), full((D + E, E)),
        full((D, D)), full((D, D)), full((D + E, 2 * E)),
        full((D, D)), full((D, D)), full((D, 2 * E)),
    ]
    out = pl.pallas_call(
        _body,
        grid=(B,),
        in_specs=in_specs,
        out_specs=pl.BlockSpec((1, N, D), lambda b: (b, 0, 0)),
        out_shape=jax.ShapeDtypeStruct((B, N, D), jnp.float32),
        scratch_shapes=[
            pltpu.VMEM((2 * E, NN), BF),
            pltpu.VMEM((N, D), jnp.float32),
        ],
        compiler_params=pltpu.CompilerParams(
            dimension_semantics=("parallel",),
        ),
    )(scaff, z, W1_0, W2_0, Wm0,
      W1_1, W2_1, Wm1,
      W1_2, W2_2, Wm2)
    return out


# speedup vs baseline: 2.0483x; 1.0013x over previous
import jax
import jax.numpy as jnp
from jax.experimental import pallas as pl
from jax.experimental.pallas import tpu as pltpu

B, N, D, E = 8, 512, 32, 8
NN = N * N
MB = 512
L = MB * N
NBLK = N // MB
BF = jnp.bfloat16


def _layer_pass(X_ref, m_ref, Wm_ref, h2T, k_rows, write_edges, residual_edges):
    def dot_blk(bi):
        Xb = X_ref[0:k_rows, pl.ds(bi * L, L)]
        return jnp.dot(Wm_ref[...], Xb, preferred_element_type=jnp.float32)

    def consume(bi, G):
        cols = [jnp.sum(jnp.maximum(G[0:D, i * N:(i + 1) * N], 0.0) * h2T,
                        axis=1, keepdims=True)
                for i in range(MB)]
        m_ref[pl.ds(bi * MB, MB), :] = jnp.concatenate(cols, axis=1).T
        if write_edges:
            eg = jnp.maximum(G[D:D + E], 0.0).astype(BF)
            if residual_edges:
                eg = X_ref[E:2 * E, pl.ds(bi * L, L)] + eg
            X_ref[E:2 * E, pl.ds(bi * L, L)] = eg

    def body(bi, Gprev):
        Gnew = dot_blk(bi)
        consume(bi - 1, Gprev)
        return Gnew

    Glast = jax.lax.fori_loop(1, NBLK, body, dot_blk(0))
    consume(NBLK - 1, Glast)


def _body(scaff_ref, z_ref,
          W1_0r, W2_0r, Wm0r,
          W1_1r, W2_1r, Wm1r,
          W1_2r, W2_2r, Wm2r,
          x_out_ref, X_ref, m_ref):
    X_ref[0:E, :] = scaff_ref[0]
    x = z_ref[0]
    h2T = jnp.dot(x, W2_0r[...], preferred_element_type=jnp.float32).T
    _layer_pass(X_ref, m_ref, Wm0r, h2T, E, True, False)
    x = jnp.maximum(jnp.dot(x, W1_0r[...], preferred_element_type=jnp.float32)
                    + m_ref[...], 0.0)
    h2T = jnp.dot(x, W2_1r[...], preferred_element_type=jnp.float32).T
    _layer_pass(X_ref, m_ref, Wm1r, h2T, 2 * E, True, True)
    x = x + jnp.maximum(jnp.dot(x, W1_1r[...], preferred_element_type=jnp.float32)
                        + m_ref[...], 0.0)
    h2T = jnp.dot(x, W2_2r[...], preferred_element_type=jnp.float32).T
    _layer_pass(X_ref, m_ref, Wm2r, h2T, 2 * E, False, False)
    x_out_ref[0] = jnp.maximum(
        jnp.dot(x, W1_2r[...], preferred_element_type=jnp.float32) + m_ref[...], 0.0)


def kernel(z, scaffold, W1_0, W2_0, Wg_0, We_0,
           W1_1, W2_1, Wge_1, Wgs_1, We1_1, We2_1,
           W1_2, W2_2, Wge_2, Wgs_2, We1_2, We2_2):
    scaff = scaffold.reshape(B, E, NN).astype(BF)
    Wm0 = jnp.concatenate([Wg_0.T, We_0.T], axis=0).astype(BF)
    Wm1 = jnp.concatenate(
        [jnp.concatenate([Wgs_1.T, Wge_1.T], axis=1),
         jnp.concatenate([We2_1.T, We1_1.T], axis=1)], axis=0).astype(BF)
    Wm2 = jnp.concatenate([Wgs_2.T, Wge_2.T], axis=1).astype(BF)

    full = lambda s: pl.BlockSpec(s, lambda b: tuple(0 for _ in s))
    in_specs = [
        pl.BlockSpec((1, E, NN), lambda b: (b, 0, 0)),
        pl.BlockSpec((1, N, D), lambda b: (b, 0, 0)),
        full((D, D)), full((D, D)), full((D + E, E)),
        full((D, D)), full((D, D)), full((D + E, 2 * E)),
        full((D, D)), full((D, D)), full((D, 2 * E)),
    ]
    out = pl.pallas_call(
        _body,
        grid=(B,),
        in_specs=in_specs,
        out_specs=pl.BlockSpec((1, N, D), lambda b: (b, 0, 0)),
        out_shape=jax.ShapeDtypeStruct((B, N, D), jnp.float32),
        scratch_shapes=[
            pltpu.VMEM((2 * E, NN), BF),
            pltpu.VMEM((N, D), jnp.float32),
        ],
        compiler_params=pltpu.CompilerParams(
            dimension_semantics=("parallel",),
        ),
    )(scaff, z, W1_0, W2_0, Wm0,
      W1_1, W2_1, Wm1,
      W1_2, W2_2, Wm2)
    return out
